# baseline (speedup 1.0000x reference)
import jax
import jax.numpy as jnp
from jax.experimental import pallas as pl

L = 2048
N_BASES = 4
BI = 256
NI = L // BI


def _body(seqrow_ref, pairsrow_ref, out8_ref, idx_ref):
    i = pl.program_id(0)
    row = seqrow_ref[:, :]
    col = jnp.transpose(seqrow_ref[:, pl.ds(i * BI, BI)], (1, 0))
    pv = jnp.transpose(pairsrow_ref[:, pl.ds(i * BI, BI)], (1, 0))
    for c in range(N_BASES):
        out8_ref[c] = jnp.broadcast_to((col == c).astype(jnp.float32), (BI, L))
        out8_ref[c + N_BASES] = jnp.broadcast_to(
            (row == c).astype(jnp.float32), (BI, L))
    jidx = jax.lax.broadcasted_iota(jnp.int32, (BI, L), 1)
    idx_ref[:, :] = (pv == jidx).astype(jnp.float32)


def kernel(seq, pairs, base_table):
    del base_table
    out8, idx = pl.pallas_call(
        _body,
        grid=(NI,),
        in_specs=[
            pl.BlockSpec((1, L), lambda i: (0, 0)),
            pl.BlockSpec((1, L), lambda i: (0, 0)),
        ],
        out_specs=[
            pl.BlockSpec((2 * N_BASES, BI, L), lambda i: (0, i, 0)),
            pl.BlockSpec((BI, L), lambda i: (i, 0)),
        ],
        out_shape=[
            jax.ShapeDtypeStruct((2 * N_BASES, L, L), jnp.float32),
            jax.ShapeDtypeStruct((L, L), jnp.float32),
        ],
    )(seq.reshape(1, L), pairs.reshape(1, L))
    return (out8.reshape(1, 2 * N_BASES, L, L), idx.reshape(1, 1, L, L))
